# Initial kernel scaffold; baseline (speedup 1.0000x reference)
import jax
import jax.numpy as jnp
from jax.experimental import pallas as pl


def kernel(hidden_states, router_w, gate_w, up_w, down_w):
    raise NotImplementedError("write your pallas kernel here")


# speedup vs baseline: 1.0307x; 1.0307x over previous
import functools

import jax
import jax.numpy as jnp
from jax.experimental import pallas as pl
from jax.experimental.pallas import tpu as pltpu

B, S, D, FF, E, TOP_K = 2, 2048, 2048, 4096, 8, 2
T = B * S


_RTM = 512


def _router_body(x_ref, rw_ref, logits_ref, combine_ref):
    x = x_ref[...]
    logits = jax.lax.dot_general(
        x, rw_ref[...], (((1,), (1,)), ((), ())),
        preferred_element_type=jnp.float32)
    logits_ref[...] = logits
    m = jnp.max(logits, axis=1, keepdims=True)
    ex = jnp.exp(logits - m)
    probs = ex / jnp.sum(ex, axis=1, keepdims=True)
    idx = jax.lax.broadcasted_iota(jnp.int32, probs.shape, 1)
    w1 = jnp.max(probs, axis=1, keepdims=True)
    e1 = jnp.min(jnp.where(probs == w1, idx, E), axis=1, keepdims=True)
    probs2 = jnp.where(idx == e1, -1.0, probs)
    w2 = jnp.max(probs2, axis=1, keepdims=True)
    e2 = jnp.min(jnp.where(probs2 == w2, idx, E), axis=1, keepdims=True)
    combine_ref[...] = (jnp.where(idx == e1, w1, 0.0)
                        + jnp.where(idx == e2, w2, 0.0))


def _run_router(x, router_w):
    return pl.pallas_call(
        _router_body,
        grid=(T // _RTM,),
        in_specs=[
            pl.BlockSpec((_RTM, D), lambda i: (i, 0)),
            pl.BlockSpec((E, D), lambda i: (0, 0)),
        ],
        out_specs=[
            pl.BlockSpec((_RTM, E), lambda i: (i, 0)),
            pl.BlockSpec((_RTM, E), lambda i: (i, 0)),
        ],
        out_shape=[
            jax.ShapeDtypeStruct((T, E), jnp.float32),
            jax.ShapeDtypeStruct((T, E), jnp.float32),
        ],
    )(x, router_w)


_TM = 1024
_TN = 512


def _ffn_body(x_ref, g_ref, u_ref, d_ref, c_ref, out_ref):
    e = pl.program_id(1)
    j = pl.program_id(2)
    x = x_ref[...]
    g = jax.lax.dot_general(x, g_ref[0], (((1,), (1,)), ((), ())),
                            preferred_element_type=jnp.float32)
    u = jax.lax.dot_general(x, u_ref[0], (((1,), (1,)), ((), ())),
                            preferred_element_type=jnp.float32)
    h = (g * jax.lax.logistic(g) * u).astype(jnp.bfloat16)
    part = jax.lax.dot_general(h, d_ref[0], (((1,), (1,)), ((), ())),
                               preferred_element_type=jnp.float32)
    eidx = jax.lax.broadcasted_iota(jnp.int32, (1, E), 1)
    c = jnp.sum(jnp.where(eidx == e, c_ref[...], 0.0), axis=1,
                keepdims=True)
    contrib = c * part

    @pl.when(jnp.logical_and(e == 0, j == 0))
    def _():
        out_ref[...] = contrib

    @pl.when(jnp.logical_not(jnp.logical_and(e == 0, j == 0)))
    def _():
        out_ref[...] += contrib


def _run_ffn(x_bf, gate_bf, up_bf, down_bf, combine):
    return pl.pallas_call(
        _ffn_body,
        grid=(T // _TM, E, FF // _TN),
        in_specs=[
            pl.BlockSpec((_TM, D), lambda i, e, j: (i, 0)),
            pl.BlockSpec((1, _TN, D), lambda i, e, j: (e, j, 0)),
            pl.BlockSpec((1, _TN, D), lambda i, e, j: (e, j, 0)),
            pl.BlockSpec((1, D, _TN), lambda i, e, j: (e, 0, j)),
            pl.BlockSpec((_TM, E), lambda i, e, j: (i, 0)),
        ],
        out_specs=pl.BlockSpec((_TM, D), lambda i, e, j: (i, 0)),
        out_shape=jax.ShapeDtypeStruct((T, D), jnp.float32),
    )(x_bf, gate_bf, up_bf, down_bf, combine)


def kernel(hidden_states, router_w, gate_w, up_w, down_w):
    x = hidden_states.reshape(T, D)
    router_logits, combine = _run_router(x, router_w)
    out = _run_ffn(x.astype(jnp.bfloat16), gate_w.astype(jnp.bfloat16),
                   up_w.astype(jnp.bfloat16), down_w.astype(jnp.bfloat16),
                   combine)
    return out.reshape(B, S, D), router_logits


# speedup vs baseline: 1.8676x; 1.8120x over previous
import functools

import jax
import jax.numpy as jnp
from jax import lax
from jax.experimental import pallas as pl
from jax.experimental.pallas import tpu as pltpu
from jax.experimental.pallas import tpu_sc as plsc

B, S, D, FF, E, TOP_K = 2, 2048, 2048, 4096, 8, 2
T = B * S
J = T * TOP_K
TM = 512
P = J + E * TM
NT = P // TM
TN = 512
L = 16


_RTM = 512


def _router_body(x_ref, rw_ref, logits_ref, w_ref, e_ref):
    x = x_ref[...]
    logits = lax.dot_general(x, rw_ref[...], (((1,), (1,)), ((), ())),
                             preferred_element_type=jnp.float32)
    logits_ref[...] = logits
    m = jnp.max(logits, axis=1, keepdims=True)
    ex = jnp.exp(logits - m)
    probs = ex / jnp.sum(ex, axis=1, keepdims=True)
    idx = lax.broadcasted_iota(jnp.int32, probs.shape, 1)
    w1 = jnp.max(probs, axis=1, keepdims=True)
    e1 = jnp.min(jnp.where(probs == w1, idx, E), axis=1, keepdims=True)
    probs2 = jnp.where(idx == e1, -1.0, probs)
    w2 = jnp.max(probs2, axis=1, keepdims=True)
    e2 = jnp.min(jnp.where(probs2 == w2, idx, E), axis=1, keepdims=True)
    w_ref[...] = jnp.concatenate([w1, w2], axis=1)
    e_ref[...] = jnp.concatenate([e1, e2], axis=1)


def _run_router(x, router_w):
    return pl.pallas_call(
        _router_body,
        grid=(T // _RTM,),
        in_specs=[
            pl.BlockSpec((_RTM, D), lambda i: (i, 0)),
            pl.BlockSpec((E, D), lambda i: (0, 0)),
        ],
        out_specs=[
            pl.BlockSpec((_RTM, E), lambda i: (i, 0)),
            pl.BlockSpec((_RTM, TOP_K), lambda i: (i, 0)),
            pl.BlockSpec((_RTM, TOP_K), lambda i: (i, 0)),
        ],
        out_shape=[
            jax.ShapeDtypeStruct((T, E), jnp.float32),
            jax.ShapeDtypeStruct((T, TOP_K), jnp.float32),
            jax.ShapeDtypeStruct((T, TOP_K), jnp.int32),
        ],
    )(x, router_w)


_NW1 = 16
_JPW = J // _NW1
_PPW1 = P // _NW1


_DSTAGE = 5


def _dispatch_body(e_hbm, w_hbm, dest_hbm, te_hbm, rt_hbm, rw_hbm,
                   ev_v, wv_v, rank_v, dest_v, dest2_v, tok2_v, w2_v,
                   te_v, cnt_smem):
    cid = lax.axis_index("c")

    @pl.when(cid == 0)
    def _core0():
        wid = lax.axis_index("s")
        base_j = wid * _JPW
        nv = _JPW // L
        lane = lax.iota(jnp.int32, L)

        @pl.when(wid == 0)
        def _():
            for e in range(E):
                cnt_smem[e] = 0

        pltpu.sync_copy(e_hbm.at[pl.ds(base_j, _JPW)], ev_v)
        pltpu.sync_copy(w_hbm.at[pl.ds(base_j, _JPW)], wv_v)
        plsc.subcore_barrier()

        cnt = [jnp.zeros((), jnp.int32) for _ in range(E)]
        for v in range(nv):
            ev = ev_v[pl.ds(v * L, L)]
            rank = jnp.zeros((L,), jnp.int32)
            for e in range(E):
                m = ev == e
                mi = jnp.where(m, 1, 0)
                incl = plsc.cumsum(mi)
                rank = rank + jnp.where(m, cnt[e] + incl - 1, 0)
                cnt[e] = cnt[e] + jnp.sum(mi)
            rank_v[pl.ds(v * L, L)] = rank

        if _DSTAGE < 2:
            pltpu.sync_copy(rank_v, dest_hbm.at[pl.ds(base_j, _JPW)])
            return

        pre_s = [plsc.fetch_and_add(cnt_smem.at[e], cnt[e], subcore_id=0)
                 for e in range(E)]
        plsc.subcore_barrier()
        tot_s = [plsc.fetch_and_add(cnt_smem.at[e], 0, subcore_id=0)
                 for e in range(E)]
        pre = jnp.zeros((L,), jnp.int32)
        tot = jnp.zeros((L,), jnp.int32)
        for e in range(E):
            pre = jnp.where(lane == e, pre_s[e], pre)
            tot = jnp.where(lane == e, tot_s[e], tot)
        ptot = ((tot + (TM - 1)) // TM) * TM
        endv = plsc.cumsum(ptot)
        off = endv - ptot
        off_pre = off + pre
        basis = [jnp.sum(jnp.where(lane == e, off_pre, 0)) for e in range(E)]

        if _DSTAGE < 3:
            hist_v[...] = off_pre
            pltpu.sync_copy(hist_v, dest_hbm.at[pl.ds(base_j, L)])
            return

        for v in range(nv):
            ev = ev_v[pl.ds(v * L, L)]
            dvec = rank_v[pl.ds(v * L, L)]
            for e in range(E):
                dvec = dvec + jnp.where(ev == e, basis[e], 0)
            dest_v[pl.ds(v * L, L)] = dvec
            r, c = (v * L) // 128, (v * L) % 128
            dest2_v[r, pl.ds(c, L)] = dvec
            tok2_v[r, pl.ds(c, L)] = (base_j + v * L + lane) & (T - 1)
            w2_v[r, pl.ds(c, L)] = wv_v[pl.ds(v * L, L)]

        pltpu.sync_copy(dest_v, dest_hbm.at[pl.ds(base_j, _JPW)])

        if _DSTAGE < 4:
            return

        for r in range(_JPW // 128):
            pltpu.sync_copy(tok2_v.at[r], rt_hbm.at[dest2_v.at[r]])
            pltpu.sync_copy(w2_v.at[r], rw_hbm.at[dest2_v.at[r]])

        if _DSTAGE < 5:
            return

        te_lo = jnp.zeros((L,), jnp.int32)
        te_hi = jnp.zeros((L,), jnp.int32)
        for e in range(E):
            end_e = jnp.sum(jnp.where(lane == e, endv, 0))
            te_lo = te_lo + jnp.where(lane * TM >= end_e, 1, 0)
            te_hi = te_hi + jnp.where((L + lane) * TM >= end_e, 1, 0)
        te_v[pl.ds(0, L)] = jnp.minimum(te_lo, E - 1)
        te_v[pl.ds(L, L)] = jnp.minimum(te_hi, E - 1)
        te_v[pl.ds(2 * L, L)] = tot
        te_v[pl.ds(3 * L, L)] = endv
        te_v[pl.ds(4 * L, L)] = off_pre
        pltpu.sync_copy(te_v, te_hbm.at[wid])


def _run_dispatch(e_cat, w_cat):
    mesh = plsc.VectorSubcoreMesh(core_axis_name="c", subcore_axis_name="s")
    return pl.kernel(
        _dispatch_body,
        out_type=[
            jax.ShapeDtypeStruct((J,), jnp.int32),
            jax.ShapeDtypeStruct((_NW1, 80), jnp.int32),
            jax.ShapeDtypeStruct((P,), jnp.int32),
            jax.ShapeDtypeStruct((P,), jnp.float32),
        ],
        mesh=mesh,
        compiler_params=pltpu.CompilerParams(needs_layout_passes=False),
        scratch_types=[
            pltpu.VMEM((_JPW,), jnp.int32),
            pltpu.VMEM((_JPW,), jnp.float32),
            pltpu.VMEM((_JPW,), jnp.int32),
            pltpu.VMEM((_JPW,), jnp.int32),
            pltpu.VMEM((_JPW // 128, 128), jnp.int32),
            pltpu.VMEM((_JPW // 128, 128), jnp.int32),
            pltpu.VMEM((_JPW // 128, 128), jnp.float32),
            pltpu.VMEM((80,), jnp.int32),
            pltpu.SMEM((E,), jnp.int32),
        ],
    )(e_cat, w_cat)


_NW = 32
_PPW = P // _NW
_GC = 48


def _gather_body(rt_hbm, x_hbm, xs_hbm, rt_v, rows_v, sem):
    wid = lax.axis_index("s") * 2 + lax.axis_index("c")
    base = wid * _PPW
    pltpu.sync_copy(rt_hbm.at[pl.ds(base, _PPW)], rt_v)
    for i in range(_PPW // L):
        rt = rt_v[pl.ds(i * L, L)]
        rt_v[pl.ds(i * L, L)] = jnp.minimum(jnp.maximum(rt, 0), T - 1)
    for c in range(_PPW // _GC):
        pltpu.async_copy(x_hbm.at[rt_v.at[pl.ds(c * _GC, _GC)]], rows_v,
                         sem).wait()
        pltpu.sync_copy(rows_v, xs_hbm.at[pl.ds(base + c * _GC, _GC)])


def _run_gather(rt, x):
    mesh = plsc.VectorSubcoreMesh(core_axis_name="c", subcore_axis_name="s")
    return pl.kernel(
        _gather_body,
        out_type=jax.ShapeDtypeStruct((P, D), jnp.float32),
        mesh=mesh,
        scratch_types=[
            pltpu.VMEM((_PPW,), jnp.int32),
            pltpu.VMEM((_GC, D), jnp.float32),
            pltpu.SemaphoreType.DMA,
        ],
    )(rt, x)


def _ffn_body(te_ref, xs_ref, g_ref, u_ref, d_ref, rw_ref, out_ref):
    j = pl.program_id(1)
    x = xs_ref[...]
    g = lax.dot_general(x, g_ref[0], (((1,), (1,)), ((), ())),
                        preferred_element_type=jnp.float32)
    u = lax.dot_general(x, u_ref[0], (((1,), (1,)), ((), ())),
                        preferred_element_type=jnp.float32)
    h = g * lax.logistic(g) * u
    part = lax.dot_general(h, d_ref[0], (((1,), (1,)), ((), ())),
                           preferred_element_type=jnp.float32)

    @pl.when(j == 0)
    def _():
        out_ref[...] = part

    @pl.when(j != 0)
    def _():
        out_ref[...] += part

    @pl.when(j == FF // TN - 1)
    def _():
        out_ref[...] = out_ref[...] * rw_ref[...]


def _run_ffn(te, xs, gate_w, up_w, down_w, rw_col):
    grid_spec = pltpu.PrefetchScalarGridSpec(
        num_scalar_prefetch=1,
        grid=(NT, FF // TN),
        in_specs=[
            pl.BlockSpec((TM, D), lambda i, j, te: (i, 0)),
            pl.BlockSpec((1, TN, D), lambda i, j, te: (te[i], j, 0)),
            pl.BlockSpec((1, TN, D), lambda i, j, te: (te[i], j, 0)),
            pl.BlockSpec((1, D, TN), lambda i, j, te: (te[i], 0, j)),
            pl.BlockSpec((TM, 1), lambda i, j, te: (i, 0)),
        ],
        out_specs=pl.BlockSpec((TM, D), lambda i, j, te: (i, 0)),
    )
    return pl.pallas_call(
        _ffn_body,
        grid_spec=grid_spec,
        out_shape=jax.ShapeDtypeStruct((P, D), jnp.float32),
        compiler_params=pltpu.CompilerParams(vmem_limit_bytes=100 * 1024 * 1024),
    )(te, xs, gate_w, up_w, down_w, rw_col)


_TPW = T // _NW
_CC = 16


def _combine_body(ys_hbm, dest_hbm, out_hbm, d0_v, d1_v, buf0_v, buf1_v, sem):
    wid = lax.axis_index("s") * 2 + lax.axis_index("c")
    t0 = wid * _TPW
    pltpu.sync_copy(dest_hbm.at[pl.ds(t0, _TPW)], d0_v)
    pltpu.sync_copy(dest_hbm.at[pl.ds(T + t0, _TPW)], d1_v)
    nvec = _CC * D // L

    for c in range(_TPW // _CC):
        cp0 = pltpu.async_copy(ys_hbm.at[d0_v.at[pl.ds(c * _CC, _CC)]],
                               buf0_v, sem)
        cp1 = pltpu.async_copy(ys_hbm.at[d1_v.at[pl.ds(c * _CC, _CC)]],
                               buf1_v, sem)
        cp0.wait()
        cp1.wait()

        def add_block(k, carry):
            for r in range(_CC):
                buf0_v[r, pl.ds(k * L, L)] = (buf0_v[r, pl.ds(k * L, L)]
                                              + buf1_v[r, pl.ds(k * L, L)])
            return carry

        lax.fori_loop(0, D // L, add_block, 0)
        pltpu.sync_copy(buf0_v, out_hbm.at[pl.ds(t0 + c * _CC, _CC)])


def _run_combine(ys, dest):
    mesh = plsc.VectorSubcoreMesh(core_axis_name="c", subcore_axis_name="s")
    return pl.kernel(
        _combine_body,
        out_type=jax.ShapeDtypeStruct((T, D), jnp.float32),
        mesh=mesh,
        compiler_params=pltpu.CompilerParams(needs_layout_passes=False),
        scratch_types=[
            pltpu.VMEM((_TPW,), jnp.int32),
            pltpu.VMEM((_TPW,), jnp.int32),
            pltpu.VMEM((_CC, D), jnp.float32),
            pltpu.VMEM((_CC, D), jnp.float32),
            pltpu.SemaphoreType.DMA,
        ],
    )(ys, dest)


def _xla_dispatch_probe(e_cat, w_cat):
    order = jnp.argsort(e_cat, stable=True)
    cnt = jnp.sum((e_cat[:, None] == jnp.arange(E)[None, :]).astype(jnp.int32),
                  axis=0)
    ptot = (cnt + TM - 1) // TM * TM
    end = jnp.cumsum(ptot)
    off = end - ptot
    cstart = jnp.cumsum(cnt) - cnt
    sorted_e = e_cat[order]
    pos = jnp.arange(J) - cstart[sorted_e]
    dest = jnp.zeros((J,), jnp.int32).at[order].set(
        (off[sorted_e] + pos).astype(jnp.int32))
    rt = jnp.zeros((P,), jnp.int32).at[dest].set(
        (jnp.arange(J) % T).astype(jnp.int32))
    rw = jnp.zeros((P,), jnp.float32).at[dest].set(w_cat)
    te = jnp.minimum(
        jnp.sum((jnp.arange(32)[:, None] * TM >= end[None, :]).astype(jnp.int32),
                axis=1), E - 1).astype(jnp.int32)
    return dest, te, rt, rw


def kernel(hidden_states, router_w, gate_w, up_w, down_w):
    x = hidden_states.reshape(T, D)
    router_logits, w_out, e_out = _run_router(x, router_w)
    e_cat = e_out.T.reshape(J)
    w_cat = w_out.T.reshape(J)
    dest, te_rows, rt, rw = _run_dispatch(e_cat, w_cat)
    te = te_rows[0, :32]
    xs = _run_gather(rt, x)
    ys = _run_ffn(te, xs, gate_w, up_w, down_w, rw.reshape(P, 1))
    out = _run_combine(ys, dest)
    return out.reshape(B, S, D), router_logits
